# Initial kernel scaffold; baseline (speedup 1.0000x reference)
import jax
import jax.numpy as jnp
from jax.experimental import pallas as pl


def kernel(x, w1p, b1p, w2p, b2p):
    raise NotImplementedError("write your pallas kernel here")


# speedup vs baseline: 1.3324x; 1.3324x over previous
import jax
import jax.numpy as jnp
from jax.experimental import pallas as pl
from jax.experimental.pallas import tpu as pltpu

_IN_F = 4
_HID = 128
_OUT_F = 2
_TB = 8192


def _round_up(n, m):
    return (n + m - 1) // m * m


def _mlp_rows_kernel(x_ref, w1t_ref, b1_ref, w2t_ref, b2_ref, o_ref):
    h = jnp.dot(x_ref[...], w1t_ref[...],
                preferred_element_type=jnp.float32)
    h = jnp.maximum(h + b1_ref[...], 0.0)
    y = jnp.dot(h, w2t_ref[...],
                preferred_element_type=jnp.float32)
    o_ref[...] = (y + b2_ref[...]).astype(o_ref.dtype)


def kernel(x, w1p, b1p, w2p, b2p):
    w1t = w1p.T
    b1r = b1p.reshape(1, _HID)
    w2t = w2p.T
    b2r = b2p.reshape(1, _OUT_F)

    B = x.shape[0]
    tb = min(_TB, _round_up(B, 8))
    b_pad = _round_up(B, tb)
    if b_pad != B:
        x = jnp.pad(x, ((0, b_pad - B), (0, 0)))

    out = pl.pallas_call(
        _mlp_rows_kernel,
        out_shape=jax.ShapeDtypeStruct((b_pad, _OUT_F), x.dtype),
        grid=(b_pad // tb,),
        in_specs=[
            pl.BlockSpec((tb, _IN_F), lambda i: (i, 0)),
            pl.BlockSpec(w1t.shape, lambda i: (0, 0)),
            pl.BlockSpec(b1r.shape, lambda i: (0, 0)),
            pl.BlockSpec(w2t.shape, lambda i: (0, 0)),
            pl.BlockSpec(b2r.shape, lambda i: (0, 0)),
        ],
        out_specs=pl.BlockSpec((tb, _OUT_F), lambda i: (i, 0)),
        compiler_params=pltpu.CompilerParams(
            dimension_semantics=("parallel",)),
    )(x, w1t, b1r, w2t, b2r)

    return out[:B] if b_pad != B else out
